# baseline (speedup 1.0000x reference)
import jax
import jax.numpy as jnp
from jax.experimental import pallas as pl

_T, _H, _W, _D = 32, 24, 24, 768


def _body(t_ref, h_ref, w_ref, o_ref):
    t = t_ref[pl.program_id(0)]
    h = h_ref[...]
    w = w_ref[...]
    hw = h[:, None, :] + w[None, :, :]
    o_ref[0, 0] = t[None, None, :] + hw


def kernel(B, T, H, W, t_table, h_table, w_table):
    out_shape = jax.ShapeDtypeStruct((1, _T, _H, _W, _D), jnp.float32)
    return pl.pallas_call(
        _body,
        grid=(_T,),
        in_specs=[
            pl.BlockSpec((_T, _D), lambda i: (0, 0)),
            pl.BlockSpec((_H, _D), lambda i: (0, 0)),
            pl.BlockSpec((_W, _D), lambda i: (0, 0)),
        ],
        out_specs=pl.BlockSpec((1, 1, _H, _W, _D), lambda i: (0, i, 0, 0, 0)),
        out_shape=out_shape,
    )(t_table, h_table, w_table)
